# Initial kernel scaffold; baseline (speedup 1.0000x reference)
import jax
import jax.numpy as jnp
from jax.experimental import pallas as pl


def kernel(x, gate_W, gate_b, base_W, base_b, lora_A, lora_B):
    raise NotImplementedError("write your pallas kernel here")


# speedup vs baseline: 3.7526x; 3.7526x over previous
import functools

import jax
import jax.numpy as jnp
from jax.experimental import pallas as pl
from jax.experimental.pallas import tpu as pltpu


def _gelu_exact(v):
    return 0.5 * v * (1.0 + jax.lax.erf(v * 0.7071067811865476))


def _mole_block(x_ref, gWt_ref, gb_ref, bWt_ref, bb_ref, Af_ref, Bf_ref,
                out_ref, probs_ref, hs_ref, *, rank, n_experts):
    j = pl.program_id(1)

    @pl.when(j == 0)
    def _router_and_lora_a():
        xb = x_ref[...]
        logits = jnp.dot(xb, gWt_ref[...], preferred_element_type=jnp.float32)
        logits = logits + gb_ref[...]
        m = jnp.max(logits, axis=-1, keepdims=True)
        ex = jnp.exp(logits - m)
        probs = ex / jnp.sum(ex, axis=-1, keepdims=True)
        probs_ref[...] = probs
        maxp = jnp.max(probs, axis=-1, keepdims=True)
        eids = jax.lax.broadcasted_iota(jnp.int32, probs.shape, 1)
        eidx = jnp.min(jnp.where(probs >= maxp, eids, n_experts),
                       axis=-1, keepdims=True)
        h = _gelu_exact(
            jnp.dot(xb, Af_ref[...], preferred_element_type=jnp.float32))
        lane_e = jax.lax.broadcasted_iota(jnp.int32, h.shape, 1) // rank
        hs_ref[...] = h * jnp.where(lane_e == eidx, maxp, 0.0)

    base = jnp.dot(x_ref[...], bWt_ref[...], preferred_element_type=jnp.float32)
    base = _gelu_exact(base + bb_ref[...])
    out_ref[...] = base + jnp.dot(hs_ref[...], Bf_ref[...],
                                  preferred_element_type=jnp.float32)


def kernel(x, gate_W, gate_b, base_W, base_b, lora_A, lora_B):
    b, s, d = x.shape
    e, _, r = lora_A.shape
    nt = b * s
    xf = x.reshape(nt, d)
    gWt = gate_W.T
    bWt = base_W.T
    Af = jnp.transpose(lora_A, (1, 0, 2)).reshape(d, e * r)
    Bf = lora_B.reshape(e * r, d)
    gb = gate_b.reshape(1, e)
    bb = base_b.reshape(1, d)

    TB = 512
    JB = 1024
    ni = nt // TB
    nj = d // JB

    out, probs = pl.pallas_call(
        functools.partial(_mole_block, rank=r, n_experts=e),
        grid=(ni, nj),
        in_specs=[
            pl.BlockSpec((TB, d), lambda i, j: (i, 0)),
            pl.BlockSpec((d, e), lambda i, j: (0, 0)),
            pl.BlockSpec((1, e), lambda i, j: (0, 0)),
            pl.BlockSpec((d, JB), lambda i, j: (0, j)),
            pl.BlockSpec((1, JB), lambda i, j: (0, j)),
            pl.BlockSpec((d, e * r), lambda i, j: (0, 0)),
            pl.BlockSpec((e * r, JB), lambda i, j: (0, j)),
        ],
        out_specs=[
            pl.BlockSpec((TB, JB), lambda i, j: (i, j)),
            pl.BlockSpec((TB, e), lambda i, j: (i, 0)),
        ],
        out_shape=[
            jax.ShapeDtypeStruct((nt, d), jnp.float32),
            jax.ShapeDtypeStruct((nt, e), jnp.float32),
        ],
        scratch_shapes=[pltpu.VMEM((TB, e * r), jnp.float32)],
        compiler_params=pltpu.CompilerParams(
            dimension_semantics=("parallel", "arbitrary"),
        ),
    )(xf, gWt, gb, bWt, bb, Af, Bf)
    return out.reshape(b, s, d), probs


# speedup vs baseline: 4.6580x; 1.2412x over previous
import functools

import jax
import jax.numpy as jnp
from jax.experimental import pallas as pl
from jax.experimental.pallas import tpu as pltpu


def _gelu_exact(v):
    return 0.5 * v * (1.0 + jax.lax.erf(v * 0.7071067811865476))


def _mole_block(x_ref, gWt_ref, gb_ref, bWt_ref, bb_ref, Af_ref, Bf_ref,
                out_ref, probs_ref, hs_ref, *, rank, n_experts):
    j = pl.program_id(1)

    @pl.when(j == 0)
    def _router_and_lora_a():
        xb = x_ref[...]
        xb_bf = xb.astype(jnp.bfloat16)
        logits = jnp.dot(xb, gWt_ref[...], preferred_element_type=jnp.float32)
        logits = logits + gb_ref[...]
        m = jnp.max(logits, axis=-1, keepdims=True)
        ex = jnp.exp(logits - m)
        probs = ex / jnp.sum(ex, axis=-1, keepdims=True)
        probs_ref[...] = probs
        maxp = jnp.max(probs, axis=-1, keepdims=True)
        eids = jax.lax.broadcasted_iota(jnp.int32, probs.shape, 1)
        eidx = jnp.min(jnp.where(probs >= maxp, eids, n_experts),
                       axis=-1, keepdims=True)
        h = _gelu_exact(
            jnp.dot(xb_bf, Af_ref[...], preferred_element_type=jnp.float32))
        lane_e = jax.lax.broadcasted_iota(jnp.int32, h.shape, 1) // rank
        hs_ref[...] = (h * jnp.where(lane_e == eidx, maxp, 0.0)).astype(
            jnp.bfloat16)

    base = jnp.dot(x_ref[...].astype(jnp.bfloat16), bWt_ref[...],
                   preferred_element_type=jnp.float32)
    base = _gelu_exact(base + bb_ref[...])
    out_ref[...] = base + jnp.dot(hs_ref[...], Bf_ref[...],
                                  preferred_element_type=jnp.float32)


def kernel(x, gate_W, gate_b, base_W, base_b, lora_A, lora_B):
    b, s, d = x.shape
    e, _, r = lora_A.shape
    nt = b * s
    xf = x.reshape(nt, d)
    gWt = gate_W.T
    bWt = base_W.T.astype(jnp.bfloat16)
    Af = jnp.transpose(lora_A, (1, 0, 2)).reshape(d, e * r).astype(
        jnp.bfloat16)
    Bf = lora_B.reshape(e * r, d).astype(jnp.bfloat16)
    gb = gate_b.reshape(1, e)
    bb = base_b.reshape(1, d)

    TB = 512
    JB = 1024
    ni = nt // TB
    nj = d // JB

    out, probs = pl.pallas_call(
        functools.partial(_mole_block, rank=r, n_experts=e),
        grid=(ni, nj),
        in_specs=[
            pl.BlockSpec((TB, d), lambda i, j: (i, 0)),
            pl.BlockSpec((d, e), lambda i, j: (0, 0)),
            pl.BlockSpec((1, e), lambda i, j: (0, 0)),
            pl.BlockSpec((d, JB), lambda i, j: (0, j)),
            pl.BlockSpec((1, JB), lambda i, j: (0, j)),
            pl.BlockSpec((d, e * r), lambda i, j: (0, 0)),
            pl.BlockSpec((e * r, JB), lambda i, j: (0, j)),
        ],
        out_specs=[
            pl.BlockSpec((TB, JB), lambda i, j: (i, j)),
            pl.BlockSpec((TB, e), lambda i, j: (i, 0)),
        ],
        out_shape=[
            jax.ShapeDtypeStruct((nt, d), jnp.float32),
            jax.ShapeDtypeStruct((nt, e), jnp.float32),
        ],
        scratch_shapes=[pltpu.VMEM((TB, e * r), jnp.bfloat16)],
        compiler_params=pltpu.CompilerParams(
            dimension_semantics=("parallel", "arbitrary"),
        ),
    )(xf, gWt, gb, bWt, bb, Af, Bf)
    return out.reshape(b, s, d), probs


# speedup vs baseline: 5.1029x; 1.0955x over previous
import jax
import jax.numpy as jnp
from jax.experimental import pallas as pl
from jax.experimental.pallas import tpu as pltpu


def _gelu_exact(v):
    return 0.5 * v * (1.0 + jax.lax.erf(v * 0.7071067811865476))


def _mole_block(x_ref, gWt_ref, gb_ref, bWt_ref, bb_ref, Af_ref, Bf_ref,
                tri_ref, exp_ref, out_ref, probs_ref):
    xb = x_ref[...]
    xbf = xb.astype(jnp.bfloat16)

    logits = jnp.dot(xb, gWt_ref[...], preferred_element_type=jnp.float32)
    logits = logits + gb_ref[...]
    m = jnp.max(logits, axis=-1, keepdims=True)
    ex = jnp.exp(logits - m)
    rinv = 1.0 / jnp.sum(ex, axis=-1, keepdims=True)
    probs_ref[...] = ex * rinv
    onehot = (logits >= m).astype(jnp.float32)
    cnt = jnp.dot(onehot, tri_ref[...], preferred_element_type=jnp.float32)
    scale8 = onehot * (cnt == 1.0).astype(jnp.float32) * rinv
    scale = jnp.dot(scale8, exp_ref[...], preferred_element_type=jnp.float32)

    h = _gelu_exact(
        jnp.dot(xbf, Af_ref[...], preferred_element_type=jnp.float32))
    hs = (h * scale).astype(jnp.bfloat16)

    base = jnp.dot(xbf, bWt_ref[...], preferred_element_type=jnp.float32)
    base = _gelu_exact(base + bb_ref[...])
    out_ref[...] = base + jnp.dot(hs, Bf_ref[...],
                                  preferred_element_type=jnp.float32)


def kernel(x, gate_W, gate_b, base_W, base_b, lora_A, lora_B):
    b, s, d = x.shape
    e, _, r = lora_A.shape
    nt = b * s
    xf = x.reshape(nt, d)
    gWt = gate_W.T
    bWt = base_W.T.astype(jnp.bfloat16)
    Af = jnp.transpose(lora_A, (1, 0, 2)).reshape(d, e * r).astype(
        jnp.bfloat16)
    Bf = lora_B.reshape(e * r, d).astype(jnp.bfloat16)
    gb = gate_b.reshape(1, e)
    bb = base_b.reshape(1, d)
    tri = jnp.triu(jnp.ones((e, e), jnp.float32))
    expand = (jnp.arange(e * r, dtype=jnp.int32)[None, :] // r
              == jnp.arange(e, dtype=jnp.int32)[:, None]).astype(jnp.float32)

    TB = 512
    ni = nt // TB

    out, probs = pl.pallas_call(
        _mole_block,
        grid=(ni,),
        in_specs=[
            pl.BlockSpec((TB, d), lambda i: (i, 0)),
            pl.BlockSpec((d, e), lambda i: (0, 0)),
            pl.BlockSpec((1, e), lambda i: (0, 0)),
            pl.BlockSpec((d, d), lambda i: (0, 0)),
            pl.BlockSpec((1, d), lambda i: (0, 0)),
            pl.BlockSpec((d, e * r), lambda i: (0, 0)),
            pl.BlockSpec((e * r, d), lambda i: (0, 0)),
            pl.BlockSpec((e, e), lambda i: (0, 0)),
            pl.BlockSpec((e, e * r), lambda i: (0, 0)),
        ],
        out_specs=[
            pl.BlockSpec((TB, d), lambda i: (i, 0)),
            pl.BlockSpec((TB, e), lambda i: (i, 0)),
        ],
        out_shape=[
            jax.ShapeDtypeStruct((nt, d), jnp.float32),
            jax.ShapeDtypeStruct((nt, e), jnp.float32),
        ],
        compiler_params=pltpu.CompilerParams(
            dimension_semantics=("arbitrary",),
        ),
    )(xf, gWt, gb, bWt, bb, Af, Bf, tri, expand)
    return out.reshape(b, s, d), probs


# speedup vs baseline: 5.1073x; 1.0009x over previous
import jax
import jax.numpy as jnp
from jax.experimental import pallas as pl
from jax.experimental.pallas import tpu as pltpu


def _gelu_exact(v):
    return 0.5 * v * (1.0 + jax.lax.erf(v * 0.7071067811865476))


def _mole_block(x_ref, gWt_ref, gb_ref, bWt_ref, bb_ref, Af_ref, Bf_ref,
                tri_ref, exp_ref, out_ref, probs_ref):
    xb = x_ref[...]
    xbf = xb.astype(jnp.bfloat16)

    logits = jnp.dot(xb, gWt_ref[...], preferred_element_type=jnp.float32)
    logits = logits + gb_ref[...]
    m = jnp.max(logits, axis=-1, keepdims=True)
    ex = jnp.exp(logits - m)
    rinv = 1.0 / jnp.sum(ex, axis=-1, keepdims=True)
    probs_ref[...] = ex * rinv
    onehot = (logits >= m).astype(jnp.float32)
    cnt = jnp.dot(onehot, tri_ref[...], preferred_element_type=jnp.float32)
    scale8 = onehot * (cnt == 1.0).astype(jnp.float32) * rinv
    scale = jnp.dot(scale8, exp_ref[...], preferred_element_type=jnp.float32)

    h = _gelu_exact(
        jnp.dot(xbf, Af_ref[...], preferred_element_type=jnp.float32))
    hs = (h * scale).astype(jnp.bfloat16)

    base = jnp.dot(xbf, bWt_ref[...], preferred_element_type=jnp.float32)
    base = _gelu_exact(base + bb_ref[...])
    out_ref[...] = base + jnp.dot(hs, Bf_ref[...],
                                  preferred_element_type=jnp.float32)


def kernel(x, gate_W, gate_b, base_W, base_b, lora_A, lora_B):
    b, s, d = x.shape
    e, _, r = lora_A.shape
    nt = b * s
    xf = x.reshape(nt, d)
    gWt = gate_W.T
    bWt = base_W.T.astype(jnp.bfloat16)
    Af = jnp.transpose(lora_A, (1, 0, 2)).reshape(d, e * r).astype(
        jnp.bfloat16)
    Bf = lora_B.reshape(e * r, d).astype(jnp.bfloat16)
    gb = gate_b.reshape(1, e)
    bb = base_b.reshape(1, d)
    tri = jnp.triu(jnp.ones((e, e), jnp.float32))
    expand = (jnp.arange(e * r, dtype=jnp.int32)[None, :] // r
              == jnp.arange(e, dtype=jnp.int32)[:, None]).astype(jnp.float32)

    TB = 512
    ni = nt // TB

    out, probs = pl.pallas_call(
        _mole_block,
        grid=(ni,),
        in_specs=[
            pl.BlockSpec((TB, d), lambda i: (i, 0)),
            pl.BlockSpec((d, e), lambda i: (0, 0)),
            pl.BlockSpec((1, e), lambda i: (0, 0)),
            pl.BlockSpec((d, d), lambda i: (0, 0)),
            pl.BlockSpec((1, d), lambda i: (0, 0)),
            pl.BlockSpec((d, e * r), lambda i: (0, 0)),
            pl.BlockSpec((e * r, d), lambda i: (0, 0)),
            pl.BlockSpec((e, e), lambda i: (0, 0)),
            pl.BlockSpec((e, e * r), lambda i: (0, 0)),
        ],
        out_specs=[
            pl.BlockSpec((TB, d), lambda i: (i, 0)),
            pl.BlockSpec((TB, e), lambda i: (i, 0)),
        ],
        out_shape=[
            jax.ShapeDtypeStruct((nt, d), jnp.float32),
            jax.ShapeDtypeStruct((nt, e), jnp.float32),
        ],
        compiler_params=pltpu.CompilerParams(
            dimension_semantics=("parallel",),
        ),
    )(xf, gWt, gb, bWt, bb, Af, Bf, tri, expand)
    return out.reshape(b, s, d), probs
